# baseline (speedup 1.0000x reference)
import functools

import jax
import jax.numpy as jnp
from jax import lax
from jax.experimental import pallas as pl
from jax.experimental.pallas import tpu as pltpu
from jax.experimental.pallas import tpu_sc as plsc


DIM = 64
CHUNK = 128
NBUF = 2


def _make_gather(num_workers: int, chunks_per_worker: int):
  mesh = plsc.VectorSubcoreMesh(core_axis_name="c", subcore_axis_name="s")
  n_rows = num_workers * chunks_per_worker * CHUNK

  @functools.partial(
      pl.kernel,
      out_type=jax.ShapeDtypeStruct((n_rows, DIM), jnp.float32),
      mesh=mesh,
      scratch_types=[
          pltpu.VMEM((chunks_per_worker, CHUNK), jnp.int32),
          pltpu.VMEM((NBUF, CHUNK, DIM), jnp.float32),
          pltpu.SemaphoreType.DMA,
          pltpu.SemaphoreType.DMA,
      ],
      compiler_params=pltpu.CompilerParams(use_tc_tiling_on_sc=False),
  )
  def gather_kernel(ids_hbm, table_hbm, out_hbm, idx_v, rows_v, sem0, sem1):
    num_cores = lax.axis_size("c")
    wid = lax.axis_index("s") * num_cores + lax.axis_index("c")
    base = wid * (chunks_per_worker * CHUNK)
    sems = (sem0, sem1)

    pltpu.sync_copy(ids_hbm.at[wid], idx_v)

    pltpu.async_copy(table_hbm.at[idx_v.at[0]], rows_v.at[0], sems[0])

    @pl.loop(0, chunks_per_worker, step=NBUF)
    def _(j):
      for b in range(NBUF):
        cur = j + b
        nxt = cur + 1

        @pl.when(nxt < chunks_per_worker)
        def _():
          pltpu.async_copy(
              table_hbm.at[idx_v.at[nxt]],
              rows_v.at[(b + 1) % NBUF],
              sems[(b + 1) % NBUF],
          )

        pltpu.make_async_copy(
            table_hbm.at[idx_v.at[cur]], rows_v.at[b], sems[b]
        ).wait()
        pltpu.sync_copy(
            rows_v.at[b], out_hbm.at[pl.ds(base + cur * CHUNK, CHUNK)]
        )

  return gather_kernel


def kernel(input_ids, word_table):
  batch, seq = input_ids.shape
  n = batch * seq
  info = plsc.get_sparse_core_info()
  num_workers = info.num_cores * info.num_subcores
  chunks_per_worker = n // (num_workers * CHUNK)
  assert chunks_per_worker * num_workers * CHUNK == n

  ids = input_ids.reshape(num_workers, chunks_per_worker, CHUNK)
  ids = ids.astype(jnp.int32)
  out = _make_gather(num_workers, chunks_per_worker)(ids, word_table)
  return out.reshape(batch, seq, DIM)


# speedup vs baseline: 1.0219x; 1.0219x over previous
import functools

import jax
import jax.numpy as jnp
from jax import lax
from jax.experimental import pallas as pl
from jax.experimental.pallas import tpu as pltpu
from jax.experimental.pallas import tpu_sc as plsc


DIM = 64
CHUNK = 512
NBUF = 2


def _make_gather(num_workers: int, chunks_per_worker: int):
  mesh = plsc.VectorSubcoreMesh(core_axis_name="c", subcore_axis_name="s")
  n_rows = num_workers * chunks_per_worker * CHUNK

  @functools.partial(
      pl.kernel,
      out_type=jax.ShapeDtypeStruct((n_rows, DIM), jnp.float32),
      mesh=mesh,
      scratch_types=[
          pltpu.VMEM((chunks_per_worker, CHUNK), jnp.int32),
          pltpu.VMEM((NBUF, CHUNK, DIM), jnp.float32),
          pltpu.SemaphoreType.DMA,
          pltpu.SemaphoreType.DMA,
      ],
      compiler_params=pltpu.CompilerParams(use_tc_tiling_on_sc=False),
  )
  def gather_kernel(ids_hbm, table_hbm, out_hbm, idx_v, rows_v, sem0, sem1):
    num_cores = lax.axis_size("c")
    wid = lax.axis_index("s") * num_cores + lax.axis_index("c")
    base = wid * (chunks_per_worker * CHUNK)
    sems = (sem0, sem1)

    pltpu.sync_copy(ids_hbm.at[wid], idx_v)

    pltpu.async_copy(table_hbm.at[idx_v.at[0]], rows_v.at[0], sems[0])

    @pl.loop(0, chunks_per_worker, step=NBUF)
    def _(j):
      for b in range(NBUF):
        cur = j + b
        nxt = cur + 1

        @pl.when(nxt < chunks_per_worker)
        def _():
          pltpu.async_copy(
              table_hbm.at[idx_v.at[nxt]],
              rows_v.at[(b + 1) % NBUF],
              sems[(b + 1) % NBUF],
          )

        pltpu.make_async_copy(
            table_hbm.at[idx_v.at[cur]], rows_v.at[b], sems[b]
        ).wait()
        pltpu.sync_copy(
            rows_v.at[b], out_hbm.at[pl.ds(base + cur * CHUNK, CHUNK)]
        )

  return gather_kernel


def kernel(input_ids, word_table):
  batch, seq = input_ids.shape
  n = batch * seq
  info = plsc.get_sparse_core_info()
  num_workers = info.num_cores * info.num_subcores
  chunks_per_worker = n // (num_workers * CHUNK)
  assert chunks_per_worker * num_workers * CHUNK == n

  ids = input_ids.reshape(num_workers, chunks_per_worker, CHUNK)
  ids = ids.astype(jnp.int32)
  out = _make_gather(num_workers, chunks_per_worker)(ids, word_table)
  return out.reshape(batch, seq, DIM)


# speedup vs baseline: 1.0236x; 1.0016x over previous
import functools

import jax
import jax.numpy as jnp
from jax import lax
from jax.experimental import pallas as pl
from jax.experimental.pallas import tpu as pltpu
from jax.experimental.pallas import tpu_sc as plsc


DIM = 64
CHUNK = 256
NBUF = 4
LOOKAHEAD = 2


def _make_gather(num_workers: int, chunks_per_worker: int):
  mesh = plsc.VectorSubcoreMesh(core_axis_name="c", subcore_axis_name="s")
  n_rows = num_workers * chunks_per_worker * CHUNK

  @functools.partial(
      pl.kernel,
      out_type=jax.ShapeDtypeStruct((n_rows, DIM), jnp.float32),
      mesh=mesh,
      scratch_types=[
          pltpu.VMEM((chunks_per_worker, CHUNK), jnp.int32),
          pltpu.VMEM((NBUF, CHUNK, DIM), jnp.float32),
          pltpu.SemaphoreType.DMA((NBUF,)),
          pltpu.SemaphoreType.DMA((NBUF,)),
      ],
      compiler_params=pltpu.CompilerParams(use_tc_tiling_on_sc=False),
  )
  def gather_kernel(ids_hbm, table_hbm, out_hbm, idx_v, rows_v, gsem, ssem):
    num_cores = lax.axis_size("c")
    wid = lax.axis_index("s") * num_cores + lax.axis_index("c")
    base = wid * (chunks_per_worker * CHUNK)

    def out_slice(c):
      return out_hbm.at[pl.ds(base + c * CHUNK, CHUNK)]

    pltpu.sync_copy(ids_hbm.at[wid], idx_v)

    for c in range(LOOKAHEAD):
      pltpu.async_copy(table_hbm.at[idx_v.at[c]], rows_v.at[c], gsem.at[c])

    @pl.loop(0, chunks_per_worker, step=NBUF)
    def _(j):
      for b in range(NBUF):
        cur = j + b
        b2 = (b + LOOKAHEAD) % NBUF
        nxt = cur + LOOKAHEAD

        pltpu.make_async_copy(
            table_hbm.at[idx_v.at[cur]], rows_v.at[b], gsem.at[b]
        ).wait()
        pltpu.async_copy(rows_v.at[b], out_slice(cur), ssem.at[b])

        @pl.when(nxt < chunks_per_worker)
        def _():
          @pl.when(nxt >= NBUF)
          def _():
            pltpu.make_async_copy(
                rows_v.at[b2], out_slice(nxt - NBUF), ssem.at[b2]
            ).wait()

          pltpu.async_copy(
              table_hbm.at[idx_v.at[nxt]], rows_v.at[b2], gsem.at[b2]
          )

    for b in range(NBUF):
      c = chunks_per_worker - NBUF + b
      pltpu.make_async_copy(rows_v.at[b % NBUF], out_slice(c), ssem.at[c % NBUF]).wait()

  return gather_kernel


def kernel(input_ids, word_table):
  batch, seq = input_ids.shape
  n = batch * seq
  info = plsc.get_sparse_core_info()
  num_workers = info.num_cores * info.num_subcores
  chunks_per_worker = n // (num_workers * CHUNK)
  assert chunks_per_worker * num_workers * CHUNK == n

  ids = input_ids.reshape(num_workers, chunks_per_worker, CHUNK)
  ids = ids.astype(jnp.int32)
  out = _make_gather(num_workers, chunks_per_worker)(ids, word_table)
  return out.reshape(batch, seq, DIM)
